# Initial kernel scaffold; baseline (speedup 1.0000x reference)
import jax
import jax.numpy as jnp
from jax.experimental import pallas as pl


def kernel(x, edge_index, edge_attr, W1_0, b1_0, W1_1, b1_1, W1_2, b1_2, W1_3, b1_3, W2_0, b2_0, W2_1, b2_1, W2_2, b2_2, W2_3, b2_3):
    raise NotImplementedError("write your pallas kernel here")


# speedup vs baseline: 1.9506x; 1.9506x over previous
import functools

import jax
import jax.numpy as jnp
from jax import lax
from jax.experimental import pallas as pl
from jax.experimental.pallas import tpu as pltpu
from jax.experimental.pallas import tpu_sc as plsc

N = 10000
NPAD = 10240
E = 320000
DP = 64
NC = 2
NS = 16
NW = NC * NS
PER_TILE = E // NW
DMA_B = 125
CH = 1000
N_DMA = CH // DMA_B
N_CH = PER_TILE // CH
ROWS_PER_TILE = NPAD // NS

_f32 = jnp.float32
_bf16 = jnp.bfloat16


def _sc_mesh():
    return plsc.VectorSubcoreMesh(core_axis_name="c", subcore_axis_name="s")


_SC_PARAMS = pltpu.CompilerParams(use_tc_tiling_on_sc=False)


def _sc_gather(xpad, row2):
    @functools.partial(
        pl.kernel,
        mesh=_sc_mesh(),
        compiler_params=_SC_PARAMS,
        out_type=jax.ShapeDtypeStruct((E, DP), _f32),
        scratch_types=[
            pltpu.VMEM((N_DMA, DMA_B), jnp.int32),
            pltpu.VMEM((CH, DP), _f32),
            pltpu.SemaphoreType.DMA,
        ],
    )
    def k(x_hbm, row_hbm, out_hbm, idx_v, rows_v, sem):
        cid = lax.axis_index("c")
        sid = lax.axis_index("s")
        wid = sid * NC + cid

        def chunk(i, carry):
            ebase = pl.multiple_of(wid * PER_TILE + i * CH, CH)
            rbase = pl.multiple_of(wid * (PER_TILE // DMA_B) + i * N_DMA, N_DMA)
            pltpu.sync_copy(row_hbm.at[pl.ds(rbase, N_DMA)], idx_v)
            descs = []
            for j in range(N_DMA):
                descs.append(
                    pltpu.async_copy(
                        x_hbm.at[idx_v.at[j]],
                        rows_v.at[pl.ds(j * DMA_B, DMA_B)],
                        sem,
                    )
                )
            for d in descs:
                d.wait()
            pltpu.sync_copy(rows_v, out_hbm.at[pl.ds(ebase, CH)])
            return carry

        lax.fori_loop(0, N_CH, chunk, 0)

    return k(xpad, row2)


def _sc_scatter(msg, col2, zeros_nd):
    @functools.partial(
        pl.kernel,
        mesh=_sc_mesh(),
        compiler_params=_SC_PARAMS,
        out_type=jax.ShapeDtypeStruct((NC * NPAD, DP), _f32),
        scratch_types=[
            pltpu.VMEM((N_DMA, DMA_B), jnp.int32),
            pltpu.VMEM((CH, DP), _f32),
            pltpu.VMEM_SHARED((NPAD, DP), _f32),
            pltpu.SemaphoreType.DMA,
        ],
    )
    def k(msg_hbm, col_hbm, z_hbm, out_hbm, idx_v, vals_v, shared, sem):
        cid = lax.axis_index("c")
        sid = lax.axis_index("s")
        wid = sid * NC + cid
        myrow = pl.multiple_of(sid * ROWS_PER_TILE, ROWS_PER_TILE)

        pltpu.sync_copy(
            z_hbm.at[pl.ds(myrow, ROWS_PER_TILE)],
            shared.at[pl.ds(myrow, ROWS_PER_TILE)],
        )
        plsc.subcore_barrier()

        def chunk(i, carry):
            ebase = pl.multiple_of(wid * PER_TILE + i * CH, CH)
            rbase = pl.multiple_of(wid * (PER_TILE // DMA_B) + i * N_DMA, N_DMA)
            pltpu.sync_copy(col_hbm.at[pl.ds(rbase, N_DMA)], idx_v)
            pltpu.sync_copy(msg_hbm.at[pl.ds(ebase, CH)], vals_v)
            for j in range(N_DMA):
                pltpu.sync_copy(
                    vals_v.at[pl.ds(j * DMA_B, DMA_B)],
                    shared.at[idx_v.at[j]],
                    add=True,
                )
            return carry

        lax.fori_loop(0, N_CH, chunk, 0)
        plsc.subcore_barrier()
        pltpu.sync_copy(
            shared.at[pl.ds(myrow, ROWS_PER_TILE)],
            out_hbm.at[pl.ds(cid * NPAD + myrow, ROWS_PER_TILE)],
        )

    return k(msg, col2, zeros_nd)


_BE = 1280


def _tc_edge_mlp(xg, eap, wx, we, b0, w1, b1, w2, b2, w3, b3):
    def body(xg_ref, ea_ref, wx_ref, we_ref, b0_ref, w1_ref, b1_ref,
             w2_ref, b2_ref, w3_ref, b3_ref, out_ref):
        xgb = xg_ref[...].astype(_bf16)
        eab = ea_ref[...].astype(_bf16)
        h = jnp.dot(xgb, wx_ref[...], preferred_element_type=_f32)
        h = h + jnp.dot(eab, we_ref[...], preferred_element_type=_f32)
        h = h + b0_ref[...]
        h = jnp.maximum(h, 0.0).astype(_bf16)
        h = jnp.dot(h, w1_ref[...], preferred_element_type=_f32) + b1_ref[...]
        h = jnp.maximum(h, 0.0).astype(_bf16)
        h = jnp.dot(h, w2_ref[...], preferred_element_type=_f32) + b2_ref[...]
        h = jnp.maximum(h, 0.0).astype(_bf16)
        out_ref[...] = (
            jnp.dot(h, w3_ref[...], preferred_element_type=_f32) + b3_ref[...]
        )

    def full(shape):
        return pl.BlockSpec(shape, lambda i: (0, 0))

    return pl.pallas_call(
        body,
        grid=(E // _BE,),
        in_specs=[
            pl.BlockSpec((_BE, DP), lambda i: (i, 0)),
            pl.BlockSpec((_BE, DP), lambda i: (i, 0)),
            full((DP, 1024)), full((DP, 1024)), full((1, 1024)),
            full((1024, 512)), full((1, 512)),
            full((512, 128)), full((1, 128)),
            full((128, DP)), full((1, DP)),
        ],
        out_specs=pl.BlockSpec((_BE, DP), lambda i: (i, 0)),
        out_shape=jax.ShapeDtypeStruct((E, DP), _f32),
    )(xg, eap, wx, we, b0, w1, b1, w2, b2, w3, b3)


_BN = 2000


def _tc_node_mlp(p0, p1, xpad, wx, wa, b0, w1, b1, w2, b2, w3, b3):
    def body(p0_ref, p1_ref, x_ref, wx_ref, wa_ref, b0_ref, w1_ref, b1_ref,
             w2_ref, b2_ref, w3_ref, b3_ref, out_ref):
        s = p0_ref[...] + p1_ref[...]
        cnt = jnp.maximum(s[:, 50:51], 1.0)
        agg = (s / cnt).astype(_bf16)
        xb = x_ref[...].astype(_bf16)
        h = jnp.dot(xb, wx_ref[...], preferred_element_type=_f32)
        h = h + jnp.dot(agg, wa_ref[...], preferred_element_type=_f32)
        h = h + b0_ref[...]
        h = jnp.maximum(h, 0.0).astype(_bf16)
        h = jnp.dot(h, w1_ref[...], preferred_element_type=_f32) + b1_ref[...]
        h = jnp.maximum(h, 0.0).astype(_bf16)
        h = jnp.dot(h, w2_ref[...], preferred_element_type=_f32) + b2_ref[...]
        h = jnp.maximum(h, 0.0).astype(_bf16)
        o = jnp.dot(h, w3_ref[...], preferred_element_type=_f32) + b3_ref[...]

        @pl.when(pl.program_id(0) == 0)
        def _init():
            out_ref[...] = jnp.zeros_like(out_ref)

        out_ref[...] += jnp.sum(o, axis=0, keepdims=True) * (1.0 / N)

    def full(shape):
        return pl.BlockSpec(shape, lambda i: (0, 0))

    return pl.pallas_call(
        body,
        grid=(N // _BN,),
        in_specs=[
            pl.BlockSpec((_BN, DP), lambda i: (i, 0)),
            pl.BlockSpec((_BN, DP), lambda i: (i, 0)),
            pl.BlockSpec((_BN, DP), lambda i: (i, 0)),
            full((DP, 1024)), full((DP, 1024)), full((1, 1024)),
            full((1024, 512)), full((1, 512)),
            full((512, 256)), full((1, 256)),
            full((256, 100)), full((1, 100)),
        ],
        out_specs=pl.BlockSpec((1, 100), lambda i: (0, 0)),
        out_shape=jax.ShapeDtypeStruct((1, 100), _f32),
    )(p0, p1, xpad, wx, wa, b0, w1, b1, w2, b2, w3, b3)


def kernel(x, edge_index, edge_attr,
           W1_0, b1_0, W1_1, b1_1, W1_2, b1_2, W1_3, b1_3,
           W2_0, b2_0, W2_1, b2_1, W2_2, b2_2, W2_3, b2_3):
    row2 = edge_index[0].astype(jnp.int32).reshape(E // DMA_B, DMA_B)
    col2 = edge_index[1].astype(jnp.int32).reshape(E // DMA_B, DMA_B)
    pad = DP - 50
    xpad = jnp.pad(x, ((0, 0), (0, pad)))
    eapad = jnp.pad(edge_attr, ((0, 0), (0, pad)))

    wx = jnp.pad(W1_0[:50], ((0, pad), (0, 0))).astype(_bf16)
    we = jnp.pad(W1_0[50:], ((0, pad), (0, 0))).astype(_bf16)
    b0 = b1_0.reshape(1, -1)
    w1 = W1_1.astype(_bf16)
    b1 = b1_1.reshape(1, -1)
    w2 = W1_2.astype(_bf16)
    b2 = b1_2.reshape(1, -1)
    w3 = jnp.pad(W1_3, ((0, 0), (0, pad))).astype(_bf16)
    b3 = jnp.pad(b1_3, (0, pad)).at[50].set(1.0).reshape(1, -1)

    xg = _sc_gather(xpad, row2)
    msg = _tc_edge_mlp(xg, eapad, wx, we, b0, w1, b1, w2, b2, w3, b3)

    zeros_nd = jnp.zeros((NPAD, DP), _f32)
    parts = _sc_scatter(msg, col2, zeros_nd)
    p0 = parts[:N]
    p1 = parts[NPAD:NPAD + N]

    ux = jnp.pad(W2_0[:50], ((0, pad), (0, 0))).astype(_bf16)
    ua = jnp.pad(W2_0[50:], ((0, pad), (0, 0))).astype(_bf16)
    d0 = b2_0.reshape(1, -1)
    u1 = W2_1.astype(_bf16)
    d1 = b2_1.reshape(1, -1)
    u2 = W2_2.astype(_bf16)
    d2 = b2_2.reshape(1, -1)
    u3 = W2_3.astype(_bf16)
    d3 = b2_3.reshape(1, -1)

    return _tc_node_mlp(p0, p1, xpad, ux, ua, d0, u1, d1, u2, d2, u3, d3)


# speedup vs baseline: 2.0748x; 1.0637x over previous
import functools

import jax
import jax.numpy as jnp
from jax import lax
from jax.experimental import pallas as pl
from jax.experimental.pallas import tpu as pltpu
from jax.experimental.pallas import tpu_sc as plsc

N = 10000
NPAD = 10240
E = 320000
DP = 64
NC = 2
NS = 16
NW = NC * NS
PER_TILE = E // NW
DMA_B = 125
CH = 1000
N_DMA = CH // DMA_B
N_CH = PER_TILE // CH
ROWS_PER_TILE = NPAD // NS

_f32 = jnp.float32
_bf16 = jnp.bfloat16


def _sc_mesh():
    return plsc.VectorSubcoreMesh(core_axis_name="c", subcore_axis_name="s")


_SC_PARAMS = pltpu.CompilerParams(use_tc_tiling_on_sc=False)


def _sc_gather(xpad, row2):
    @functools.partial(
        pl.kernel,
        mesh=_sc_mesh(),
        compiler_params=_SC_PARAMS,
        out_type=jax.ShapeDtypeStruct((E, DP), _f32),
        scratch_types=[
            pltpu.VMEM((N_DMA, DMA_B), jnp.int32),
            pltpu.VMEM((CH, DP), _f32),
            pltpu.SemaphoreType.DMA,
        ],
    )
    def k(x_hbm, row_hbm, out_hbm, idx_v, rows_v, sem):
        cid = lax.axis_index("c")
        sid = lax.axis_index("s")
        wid = sid * NC + cid

        def chunk(i, carry):
            ebase = pl.multiple_of(wid * PER_TILE + i * CH, CH)
            rbase = pl.multiple_of(wid * (PER_TILE // DMA_B) + i * N_DMA, N_DMA)
            pltpu.sync_copy(row_hbm.at[pl.ds(rbase, N_DMA)], idx_v)
            descs = []
            for j in range(N_DMA):
                descs.append(
                    pltpu.async_copy(
                        x_hbm.at[idx_v.at[j]],
                        rows_v.at[pl.ds(j * DMA_B, DMA_B)],
                        sem,
                    )
                )
            for d in descs:
                d.wait()
            pltpu.sync_copy(rows_v, out_hbm.at[pl.ds(ebase, CH)])
            return carry

        lax.fori_loop(0, N_CH, chunk, 0)

    return k(xpad, row2)


def _sc_scatter(msg, col2, zeros_nd):
    @functools.partial(
        pl.kernel,
        mesh=_sc_mesh(),
        compiler_params=_SC_PARAMS,
        out_type=jax.ShapeDtypeStruct((NC * NPAD, DP), _f32),
        scratch_types=[
            pltpu.VMEM((N_DMA, DMA_B), jnp.int32),
            pltpu.VMEM((CH, DP), _f32),
            pltpu.VMEM_SHARED((NPAD, DP), _f32),
            pltpu.SemaphoreType.DMA,
        ],
    )
    def k(msg_hbm, col_hbm, z_hbm, out_hbm, idx_v, vals_v, shared, sem):
        cid = lax.axis_index("c")
        sid = lax.axis_index("s")
        wid = sid * NC + cid
        myrow = pl.multiple_of(sid * ROWS_PER_TILE, ROWS_PER_TILE)

        pltpu.sync_copy(
            z_hbm.at[pl.ds(myrow, ROWS_PER_TILE)],
            shared.at[pl.ds(myrow, ROWS_PER_TILE)],
        )
        plsc.subcore_barrier()

        def chunk(i, carry):
            ebase = pl.multiple_of(wid * PER_TILE + i * CH, CH)
            rbase = pl.multiple_of(wid * (PER_TILE // DMA_B) + i * N_DMA, N_DMA)
            pltpu.sync_copy(col_hbm.at[pl.ds(rbase, N_DMA)], idx_v)
            pltpu.sync_copy(msg_hbm.at[pl.ds(ebase, CH)], vals_v)
            for j in range(N_DMA):
                pltpu.sync_copy(
                    vals_v.at[pl.ds(j * DMA_B, DMA_B)],
                    shared.at[idx_v.at[j]],
                    add=True,
                )
            return carry

        lax.fori_loop(0, N_CH, chunk, 0)
        plsc.subcore_barrier()
        pltpu.sync_copy(
            shared.at[pl.ds(myrow, ROWS_PER_TILE)],
            out_hbm.at[pl.ds(cid * NPAD + myrow, ROWS_PER_TILE)],
        )

    return k(msg, col2, zeros_nd)


_BE = 2560


def _tc_edge_mlp(xg, eap, wx, we, b0, w1, b1, w2, b2, w3, b3):
    def body(xg_ref, ea_ref, wx_ref, we_ref, b0_ref, w1_ref, b1_ref,
             w2_ref, b2_ref, w3_ref, b3_ref, out_ref):
        xgb = xg_ref[...].astype(_bf16)
        eab = ea_ref[...].astype(_bf16)
        h = jnp.dot(xgb, wx_ref[...], preferred_element_type=_f32)
        h = h + jnp.dot(eab, we_ref[...], preferred_element_type=_f32)
        h = h + b0_ref[...]
        h = jnp.maximum(h, 0.0).astype(_bf16)
        h = jnp.dot(h, w1_ref[...], preferred_element_type=_f32) + b1_ref[...]
        h = jnp.maximum(h, 0.0).astype(_bf16)
        h = jnp.dot(h, w2_ref[...], preferred_element_type=_f32) + b2_ref[...]
        h = jnp.maximum(h, 0.0).astype(_bf16)
        out_ref[...] = (
            jnp.dot(h, w3_ref[...], preferred_element_type=_f32) + b3_ref[...]
        )

    def full(shape):
        return pl.BlockSpec(shape, lambda i: (0, 0))

    return pl.pallas_call(
        body,
        grid=(E // _BE,),
        in_specs=[
            pl.BlockSpec((_BE, DP), lambda i: (i, 0)),
            pl.BlockSpec((_BE, 50), lambda i: (i, 0)),
            full((DP, 1024)), full((50, 1024)), full((1, 1024)),
            full((1024, 512)), full((1, 512)),
            full((512, 128)), full((1, 128)),
            full((128, DP)), full((1, DP)),
        ],
        out_specs=pl.BlockSpec((_BE, DP), lambda i: (i, 0)),
        out_shape=jax.ShapeDtypeStruct((E, DP), _f32),
    )(xg, eap, wx, we, b0, w1, b1, w2, b2, w3, b3)


_BN = 2000


def _tc_node_mlp(parts3, xpad, wx, wa, b0, w1, b1, w2, b2, w3, b3):
    def body(p0_ref, p1_ref, x_ref, wx_ref, wa_ref, b0_ref, w1_ref, b1_ref,
             w2_ref, b2_ref, w3_ref, b3_ref, out_ref):
        s = p0_ref[0] + p1_ref[0]
        cnt = jnp.maximum(s[:, 50:51], 1.0)
        agg = (s / cnt).astype(_bf16)
        xb = x_ref[...].astype(_bf16)
        h = jnp.dot(xb, wx_ref[...], preferred_element_type=_f32)
        h = h + jnp.dot(agg, wa_ref[...], preferred_element_type=_f32)
        h = h + b0_ref[...]
        h = jnp.maximum(h, 0.0).astype(_bf16)
        h = jnp.dot(h, w1_ref[...], preferred_element_type=_f32) + b1_ref[...]
        h = jnp.maximum(h, 0.0).astype(_bf16)
        h = jnp.dot(h, w2_ref[...], preferred_element_type=_f32) + b2_ref[...]
        h = jnp.maximum(h, 0.0).astype(_bf16)
        o = jnp.dot(h, w3_ref[...], preferred_element_type=_f32) + b3_ref[...]

        @pl.when(pl.program_id(0) == 0)
        def _init():
            out_ref[...] = jnp.zeros_like(out_ref)

        out_ref[...] += jnp.sum(o, axis=0, keepdims=True) * (1.0 / N)

    def full(shape):
        return pl.BlockSpec(shape, lambda i: (0, 0))

    return pl.pallas_call(
        body,
        grid=(N // _BN,),
        in_specs=[
            pl.BlockSpec((1, _BN, DP), lambda i: (0, i, 0)),
            pl.BlockSpec((1, _BN, DP), lambda i: (1, i, 0)),
            pl.BlockSpec((_BN, DP), lambda i: (i, 0)),
            full((DP, 1024)), full((DP, 1024)), full((1, 1024)),
            full((1024, 512)), full((1, 512)),
            full((512, 256)), full((1, 256)),
            full((256, 100)), full((1, 100)),
        ],
        out_specs=pl.BlockSpec((1, 100), lambda i: (0, 0)),
        out_shape=jax.ShapeDtypeStruct((1, 100), _f32),
    )(parts3, parts3, xpad, wx, wa, b0, w1, b1, w2, b2, w3, b3)


def kernel(x, edge_index, edge_attr,
           W1_0, b1_0, W1_1, b1_1, W1_2, b1_2, W1_3, b1_3,
           W2_0, b2_0, W2_1, b2_1, W2_2, b2_2, W2_3, b2_3):
    row2 = edge_index[0].astype(jnp.int32).reshape(E // DMA_B, DMA_B)
    col2 = edge_index[1].astype(jnp.int32).reshape(E // DMA_B, DMA_B)
    pad = DP - 50
    xpad = jnp.pad(x, ((0, 0), (0, pad)))

    wx = jnp.pad(W1_0[:50], ((0, pad), (0, 0))).astype(_bf16)
    we = W1_0[50:].astype(_bf16)
    b0 = b1_0.reshape(1, -1)
    w1 = W1_1.astype(_bf16)
    b1 = b1_1.reshape(1, -1)
    w2 = W1_2.astype(_bf16)
    b2 = b1_2.reshape(1, -1)
    w3 = jnp.pad(W1_3, ((0, 0), (0, pad))).astype(_bf16)
    b3 = jnp.pad(b1_3, (0, pad)).at[50].set(1.0).reshape(1, -1)

    xg = _sc_gather(xpad, row2)
    msg = _tc_edge_mlp(xg, edge_attr, wx, we, b0, w1, b1, w2, b2, w3, b3)

    zeros_nd = jnp.zeros((NPAD, DP), _f32)
    parts3 = _sc_scatter(msg, col2, zeros_nd).reshape(NC, NPAD, DP)

    ux = jnp.pad(W2_0[:50], ((0, pad), (0, 0))).astype(_bf16)
    ua = jnp.pad(W2_0[50:], ((0, pad), (0, 0))).astype(_bf16)
    d0 = b2_0.reshape(1, -1)
    u1 = W2_1.astype(_bf16)
    d1 = b2_1.reshape(1, -1)
    u2 = W2_2.astype(_bf16)
    d2 = b2_2.reshape(1, -1)
    u3 = W2_3.astype(_bf16)
    d3 = b2_3.reshape(1, -1)

    return _tc_node_mlp(parts3, xpad, ux, ua, d0, u1, d1, u2, d2, u3, d3)
